# baseline (speedup 1.0000x reference)
import functools

import jax
import jax.numpy as jnp
from jax import lax
from jax.experimental import pallas as pl
from jax.experimental.pallas import tpu as pltpu
from jax.experimental.pallas import tpu_sc as plsc

D = 128
CHUNK = 128
NC = 2
NS = 16
NW = NC * NS
EPS = 1e-12


def _gather_body(n_chunks, table_hbm, idx_hbm, out_hbm, idx_v, rows_v, gsem, osem):
    spc = 512 // CHUNK
    wid = lax.axis_index("s") * NC + lax.axis_index("c")
    base = wid * n_chunks
    pltpu.sync_copy(idx_hbm.at[pl.ds(base, n_chunks)], idx_v)

    def out_at(g):
        return out_hbm.at[g // spc, pl.ds((g % spc) * CHUNK, CHUNK)]

    pltpu.async_copy(table_hbm.at[idx_v.at[0]], rows_v.at[0], gsem.at[0])

    def body(j, carry):
        b = j % 2
        nb = 1 - b

        @pl.when(j >= 1)
        def _():
            pltpu.make_async_copy(
                rows_v.at[nb], out_at(base + j - 1), osem.at[nb]
            ).wait()

        @pl.when(j + 1 < n_chunks)
        def _():
            pltpu.async_copy(
                table_hbm.at[idx_v.at[j + 1]], rows_v.at[nb], gsem.at[nb]
            )

        pltpu.make_async_copy(
            table_hbm.at[idx_v.at[j]], rows_v.at[b], gsem.at[b]
        ).wait()
        pltpu.async_copy(rows_v.at[b], out_at(base + j), osem.at[b])
        return carry

    lax.fori_loop(0, n_chunks, body, 0)
    last = (n_chunks - 1) % 2
    pltpu.make_async_copy(
        rows_v.at[last], out_at(base + n_chunks - 1), osem.at[last]
    ).wait()


def _sc_gather(table, idx2d):
    n_rows = idx2d.shape[0]
    n_chunks = n_rows // NW
    bk = (n_rows * CHUNK) // 512
    mesh = plsc.VectorSubcoreMesh(core_axis_name="c", subcore_axis_name="s")
    f = pl.kernel(
        functools.partial(_gather_body, n_chunks),
        out_type=jax.ShapeDtypeStruct((bk, 512, D), jnp.float32),
        mesh=mesh,
        scratch_types=[
            pltpu.VMEM((n_chunks, CHUNK), jnp.int32),
            pltpu.VMEM((2, CHUNK, D), jnp.float32),
            pltpu.SemaphoreType.DMA((2,)),
            pltpu.SemaphoreType.DMA((2,)),
        ],
    )
    return f(table, idx2d)


def _ln_body_first(tok_ref, seg_ref, pps_ref, gamma_ref, beta_ref, out_ref):
    _ln_compute(tok_ref, seg_ref, pps_ref, gamma_ref, beta_ref, out_ref)


def _ln_body_alias(prev_ref, tok_ref, seg_ref, pps_ref, gamma_ref, beta_ref,
                   out_ref):
    del prev_ref
    _ln_compute(tok_ref, seg_ref, pps_ref, gamma_ref, beta_ref, out_ref)


def _ln_compute(tok_ref, seg_ref, pps_ref, gamma_ref, beta_ref, out_ref):
    h = tok_ref[...]
    segb = seg_ref[...]
    pps = pps_ref[...]
    sel = segb == 1
    h = h + jnp.where(sel, pps[1][None], pps[0][None])
    mean = jnp.mean(h, axis=-1, keepdims=True)
    c = h - mean
    var = jnp.mean(c * c, axis=-1, keepdims=True)
    out_ref[...] = (gamma_ref[...] * c) / (jnp.sqrt(var) + EPS) + beta_ref[...]


BB = 8


def _tc_ln_slice(prev, tok, seg, pps, gamma, beta, blk0, B, S):
    Bk = tok.shape[0]
    grid = (Bk // BB,)
    common_in = [
        pl.BlockSpec((BB, S, D), lambda i: (i, 0, 0)),
        pl.BlockSpec((BB, S, 1), lambda i: (i, 0, 0)),
        pl.BlockSpec((2, S, D), lambda i: (0, 0, 0)),
        pl.BlockSpec((D,), lambda i: (0,)),
        pl.BlockSpec((D,), lambda i: (0,)),
    ]
    out_spec = pl.BlockSpec((BB, S, D), lambda i: (blk0 + i, 0, 0))
    out_shape = jax.ShapeDtypeStruct((B, S, D), jnp.float32)
    if prev is None:
        return pl.pallas_call(
            _ln_body_first, grid=grid, in_specs=common_in,
            out_specs=out_spec, out_shape=out_shape,
        )(tok, seg, pps, gamma, beta)
    prev_spec = pl.BlockSpec((BB, S, D), lambda i: (0, 0, 0))
    return pl.pallas_call(
        _ln_body_alias, grid=grid, in_specs=[prev_spec] + common_in,
        out_specs=out_spec, out_shape=out_shape,
        input_output_aliases={0: 0},
    )(prev, tok, seg, pps, gamma, beta)


K = 4


def kernel(x, segment_info, tok_table, pos_embedding, seg_table, gamma, beta):
    B, S = x.shape
    n_rows = (B * S) // CHUNK
    idx2d = x.reshape(n_rows, CHUNK).astype(jnp.int32)
    pps = pos_embedding[0][None] + seg_table[:, None, :]
    seg3 = segment_info.astype(jnp.int32)[..., None]

    Bk = B // K
    rk = n_rows // K
    rows = [_sc_gather(tok_table, idx2d[k * rk:(k + 1) * rk])
            for k in range(K)]
    out = None
    for k in range(K):
        tok_k = rows[k]
        seg_k = seg3[k * Bk:(k + 1) * Bk]
        out = _tc_ln_slice(out, tok_k, seg_k, pps, gamma, beta,
                           k * (Bk // BB), B, S)
    return out


# speedup vs baseline: 1.4482x; 1.4482x over previous
import functools

import jax
import jax.numpy as jnp
from jax import lax
from jax.experimental import pallas as pl
from jax.experimental.pallas import tpu as pltpu
from jax.experimental.pallas import tpu_sc as plsc

D = 128
CHUNK = 128
NC = 2
NS = 16
NW = NC * NS
EPS = 1e-12


def _gather_body(n_chunks, table_hbm, idx_hbm, out_hbm, idx_v, rows_v, gsem, osem):
    spc = 512 // CHUNK
    wid = lax.axis_index("s") * NC + lax.axis_index("c")
    base = wid * n_chunks
    pltpu.sync_copy(idx_hbm.at[pl.ds(base, n_chunks)], idx_v)

    def out_at(g):
        return out_hbm.at[g // spc, pl.ds((g % spc) * CHUNK, CHUNK)]

    pltpu.async_copy(table_hbm.at[idx_v.at[0]], rows_v.at[0], gsem.at[0])

    def body(j, carry):
        b = j % 2
        nb = 1 - b

        @pl.when(j >= 1)
        def _():
            pltpu.make_async_copy(
                rows_v.at[nb], out_at(base + j - 1), osem.at[nb]
            ).wait()

        @pl.when(j + 1 < n_chunks)
        def _():
            pltpu.async_copy(
                table_hbm.at[idx_v.at[j + 1]], rows_v.at[nb], gsem.at[nb]
            )

        pltpu.make_async_copy(
            table_hbm.at[idx_v.at[j]], rows_v.at[b], gsem.at[b]
        ).wait()
        pltpu.async_copy(rows_v.at[b], out_at(base + j), osem.at[b])
        return carry

    lax.fori_loop(0, n_chunks, body, 0)
    last = (n_chunks - 1) % 2
    pltpu.make_async_copy(
        rows_v.at[last], out_at(base + n_chunks - 1), osem.at[last]
    ).wait()


def _sc_gather(table, idx2d):
    n_rows = idx2d.shape[0]
    n_chunks = n_rows // NW
    bk = (n_rows * CHUNK) // 512
    mesh = plsc.VectorSubcoreMesh(core_axis_name="c", subcore_axis_name="s")
    f = pl.kernel(
        functools.partial(_gather_body, n_chunks),
        out_type=jax.ShapeDtypeStruct((bk, 512, D), jnp.float32),
        mesh=mesh,
        scratch_types=[
            pltpu.VMEM((n_chunks, CHUNK), jnp.int32),
            pltpu.VMEM((2, CHUNK, D), jnp.float32),
            pltpu.SemaphoreType.DMA((2,)),
            pltpu.SemaphoreType.DMA((2,)),
        ],
    )
    return f(table, idx2d)


def _ln_body_first(tok_ref, seg_ref, pps_ref, gamma_ref, beta_ref, out_ref):
    _ln_compute(tok_ref, seg_ref, pps_ref, gamma_ref, beta_ref, out_ref)


def _ln_body_alias(prev_ref, tok_ref, seg_ref, pps_ref, gamma_ref, beta_ref,
                   out_ref):
    del prev_ref
    _ln_compute(tok_ref, seg_ref, pps_ref, gamma_ref, beta_ref, out_ref)


def _ln_compute(tok_ref, seg_ref, pps_ref, gamma_ref, beta_ref, out_ref):
    h = tok_ref[...]
    segf = seg_ref[...]
    pps = pps_ref[...]
    segb = lax.broadcast_in_dim(segf, segf.shape + (1,), (0, 1))
    h = h + pps[0][None] + segb * (pps[1] - pps[0])[None]
    mean = jnp.mean(h, axis=-1, keepdims=True)
    c = h - mean
    var = jnp.mean(c * c, axis=-1, keepdims=True)
    out_ref[...] = (gamma_ref[...] * c) / (jnp.sqrt(var) + EPS) + beta_ref[...]


BB = 8


def _tc_ln_slice(prev, tok, seg, pps, gamma, beta, blk0, B, S):
    Bk = tok.shape[0]
    grid = (Bk // BB,)
    common_in = [
        pl.BlockSpec((BB, S, D), lambda i: (i, 0, 0)),
        pl.BlockSpec((BB, S), lambda i: (i, 0)),
        pl.BlockSpec((2, S, D), lambda i: (0, 0, 0)),
        pl.BlockSpec((D,), lambda i: (0,)),
        pl.BlockSpec((D,), lambda i: (0,)),
    ]
    out_spec = pl.BlockSpec((BB, S, D), lambda i: (blk0 + i, 0, 0))
    out_shape = jax.ShapeDtypeStruct((B, S, D), jnp.float32)
    if prev is None:
        return pl.pallas_call(
            _ln_body_first, grid=grid, in_specs=common_in,
            out_specs=out_spec, out_shape=out_shape,
        )(tok, seg, pps, gamma, beta)
    prev_spec = pl.BlockSpec((BB, S, D), lambda i: (0, 0, 0))
    return pl.pallas_call(
        _ln_body_alias, grid=grid, in_specs=[prev_spec] + common_in,
        out_specs=out_spec, out_shape=out_shape,
        input_output_aliases={0: 0},
    )(prev, tok, seg, pps, gamma, beta)


K = 4


def kernel(x, segment_info, tok_table, pos_embedding, seg_table, gamma, beta):
    B, S = x.shape
    n_rows = (B * S) // CHUNK
    idx2d = x.reshape(n_rows, CHUNK).astype(jnp.int32)
    pps = pos_embedding[0][None] + seg_table[:, None, :]
    seg3 = segment_info.astype(jnp.float32)

    Bk = B // K
    rk = n_rows // K
    rows = [_sc_gather(tok_table, idx2d[k * rk:(k + 1) * rk])
            for k in range(K)]
    out = None
    for k in range(K):
        tok_k = rows[k]
        seg_k = seg3[k * Bk:(k + 1) * Bk]
        out = _tc_ln_slice(out, tok_k, seg_k, pps, gamma, beta,
                           k * (Bk // BB), B, S)
    return out
